# baseline (speedup 1.0000x reference)
import functools

import jax
import jax.numpy as jnp
from jax import lax
from jax.experimental import pallas as pl
from jax.experimental.pallas import tpu as pltpu
from jax.experimental.pallas import tpu_sc as plsc

_NC = 2
_NS = 16
_NW = _NC * _NS

_GSZ = 128


@functools.partial(jax.jit, static_argnames=("d",))
def _sc_detranspose(tt, side, d):
    D, V = tt.shape
    assert d == D and D == 32
    nfull = V // 128
    rem = V - nfull * 128
    per_w = nfull // _NW
    extra = nfull - per_w * _NW
    mesh = plsc.VectorSubcoreMesh(core_axis_name="c", subcore_axis_name="s")

    import numpy as np

    ivflat = jnp.asarray(
        np.array(
            [
                g * 16 * D + l * D + c
                for c in range(D)
                for g in range(8)
                for l in range(16)
            ],
            np.int32,
        )
    )
    bwords = 128 * D

    @functools.partial(
        pl.kernel,
        out_type=jax.ShapeDtypeStruct((V * D,), jnp.float32),
        mesh=mesh,
        compiler_params=pltpu.CompilerParams(
            use_tc_tiling_on_sc=True, needs_layout_passes=False
        ),
        scratch_types=[
            pltpu.VMEM((D * 8 * 16,), jnp.int32),
            pltpu.VMEM((rem * D if rem else 8,), jnp.float32),
            pltpu.VMEM((D, 128), jnp.float32),
            pltpu.VMEM((D, 128), jnp.float32),
            pltpu.VMEM((bwords,), jnp.float32),
            pltpu.VMEM((bwords,), jnp.float32),
            pltpu.SemaphoreType.DMA,
            pltpu.SemaphoreType.DMA,
            pltpu.SemaphoreType.DMA,
            pltpu.SemaphoreType.DMA,
        ],
    )
    def k(tt_hbm, iv_hbm, side_hbm, out_hbm, iv_v, side_v,
          ib0, ib1, ob0, ob1, gi0, gi1, go0, go1):
        wid = lax.axis_index("s") * _NC + lax.axis_index("c")
        ibs, obs = (ib0, ib1), (ob0, ob1)
        gis, gos = (gi0, gi1), (go0, go1)
        pltpu.sync_copy(iv_hbm, iv_v)
        nblk = per_w + 1

        def blk_of(j):
            return wid + _NW * j

        def in_copy_start(j, b):
            for r in range(D // 8):
                pltpu.async_copy(
                    tt_hbm.at[pl.ds(r * 8, 8), pl.ds(blk_of(j) * 128, 128)],
                    ibs[b].at[pl.ds(r * 8, 8)],
                    gis[b],
                )

        def in_copy_wait(b):
            pltpu.make_async_copy(
                tt_hbm.at[:, pl.ds(0, 128)], ibs[b], gis[b]
            ).wait()

        def out_copy(j, b):
            return pltpu.make_async_copy(
                obs[b], out_hbm.at[pl.ds(blk_of(j) * bwords, bwords)], gos[b]
            )

        def transpose_block(ib, ob):
            for c in range(D):
                row = ib.at[c]
                for g in range(8):
                    v = row[pl.ds(g * 16, 16)]
                    iv = iv_v[pl.ds((c * 8 + g) * 16, 16)]
                    plsc.store_scatter(ob, [iv], v)

        def guard(j):
            return blk_of(j) < nfull

        @pl.when(guard(0))
        def _():
            in_copy_start(0, 0)

        @pl.loop(0, nblk)
        def _(j):
            b = lax.rem(j, 2)
            for bs in range(2):

                @pl.when((b == bs) & guard(j))
                def _():
                    in_copy_wait(bs)

                    @pl.when(guard(j + 1))
                    def _():
                        in_copy_start(j + 1, 1 - bs)

                    @pl.when(j >= 2)
                    def _():
                        out_copy(j - 2, bs).wait()

                    transpose_block(ibs[bs], obs[bs])
                    out_copy(j, bs).start()

        nb = jnp.where(wid < extra, per_w + 1, per_w)
        for tail_off in (2, 1):
            jj = nb - tail_off
            for bs in range(2):

                @pl.when(lax.rem(jj, 2) == bs)
                def _():
                    out_copy(jj, bs).wait()

        if rem:

            @pl.when(wid == 0)
            def _():
                pltpu.sync_copy(side_hbm, side_v)
                pltpu.sync_copy(
                    side_v, out_hbm.at[pl.ds(nfull * bwords, rem * D)]
                )


    return k(tt, ivflat, side)


@functools.partial(jax.jit, static_argnames=("n_g", "ch", "d"))
def _sc_gather(table2, idx3, n_g, ch, d):
    pack = 128 // d
    n_ch = n_g // ch
    per_w = n_g * _GSZ
    rows_per_ch = ch * _GSZ
    lines_per_ch = rows_per_ch // pack
    mesh = plsc.VectorSubcoreMesh(core_axis_name="c", subcore_axis_name="s")

    @functools.partial(
        pl.kernel,
        out_type=jax.ShapeDtypeStruct((_NW * per_w, d), jnp.float32),
        mesh=mesh,
        compiler_params=pltpu.CompilerParams(use_tc_tiling_on_sc=False),
        scratch_types=[
            pltpu.VMEM((n_g, _GSZ), jnp.int32),
            pltpu.VMEM((rows_per_ch, d), jnp.float32),
            pltpu.VMEM((rows_per_ch, d), jnp.float32),
            pltpu.SemaphoreType.DMA,
            pltpu.SemaphoreType.DMA,
            pltpu.SemaphoreType.DMA,
            pltpu.SemaphoreType.DMA,
        ],
    )
    def k(table_hbm, idx_hbm, out_hbm, idx_v, buf0, buf1, g0, g1, o0, o1):
        table_rows = table_hbm
        wid = lax.axis_index("s") * _NC + lax.axis_index("c")
        base = wid * per_w // pack
        pltpu.sync_copy(idx_hbm.at[wid], idx_v)
        bufs = (buf0, buf1)
        gsems = (g0, g1)
        osems = (o0, o1)

        def fire(c, b):
            for j in range(ch):
                pltpu.async_copy(
                    table_rows.at[idx_v.at[c * ch + j]],
                    bufs[b].at[pl.ds(j * _GSZ, _GSZ)],
                    gsems[b],
                )

        def drain_gathers(b):
            pltpu.make_async_copy(
                out_hbm.at[pl.ds(0, rows_per_ch)],
                bufs[b],
                gsems[b],
            ).wait()

        def out_copy(c, b):
            return pltpu.make_async_copy(
                bufs[b],
                out_hbm.at[pl.ds(base + c * rows_per_ch, rows_per_ch)],
                osems[b],
            )

        fire(0, 0)

        @pl.loop(0, n_ch // 2)
        def _(t):
            for b in range(2):
                c = t * 2 + b
                nb = 1 - b
                drain_gathers(b)
                out_copy(c, b).start()

                @pl.when(c >= 1)
                def _():
                    out_copy(c - 1, nb).wait()

                @pl.when(c + 1 < n_ch)
                def _():
                    fire(c + 1, nb)

        out_copy(n_ch - 1, (n_ch - 1) % 2).wait()

    return k(table2, idx3)


def kernel(x, table):
    if x.ndim != 4:
        raise ValueError(f"TokenEmbedding expects 4D input [B, H, W, C]. Got: {x.shape}")
    V, D = table.shape
    if x.shape[-1] == V:
        idx = jnp.argmax(x, axis=-1).astype(jnp.int32)
    else:
        idx = x.astype(jnp.int32)
    B, H, W = x.shape[0], x.shape[1], x.shape[2]
    N = B * H * W * (1 if x.shape[-1] == V else x.shape[-1])
    flat = idx.reshape(N)

    assert N % (_NW * _GSZ) == 0, (N,)
    assert 128 % D == 0, (D,)
    n_g = N // (_NW * _GSZ)
    ch = 1
    for cand in range(2, 9):
        if n_g % cand == 0 and (n_g // cand) % 2 == 0:
            ch = cand
    idx3 = flat.reshape(_NW, n_g, _GSZ)
    rem = V % 128
    side = table[V - rem:, :].reshape(rem * D) if rem else (
        jnp.zeros((8,), jnp.float32))
    table_lin = _sc_detranspose(table.T, side, D).reshape(V, D)
    out = _sc_gather(table_lin, idx3, n_g, ch, D)
    return out.reshape(B, H, W, D)


# speedup vs baseline: 1.2413x; 1.2413x over previous
import functools

import jax
import jax.numpy as jnp
from jax import lax
from jax.experimental import pallas as pl
from jax.experimental.pallas import tpu as pltpu
from jax.experimental.pallas import tpu_sc as plsc

_NC = 2
_NS = 16
_NW = _NC * _NS

_GSZ = 128


@functools.partial(jax.jit, static_argnames=("n_g", "ch", "d"))
def _sc_gather(table2, idx3, n_g, ch, d):
    pack = 128 // d
    n_ch = n_g // ch
    per_w = n_g * _GSZ
    rows_per_ch = ch * _GSZ
    lines_per_ch = rows_per_ch // pack
    mesh = plsc.VectorSubcoreMesh(core_axis_name="c", subcore_axis_name="s")

    @functools.partial(
        pl.kernel,
        out_type=jax.ShapeDtypeStruct((_NW * per_w, d), jnp.float32),
        mesh=mesh,
        compiler_params=pltpu.CompilerParams(use_tc_tiling_on_sc=False),
        scratch_types=[
            pltpu.VMEM((n_g, _GSZ), jnp.int32),
            pltpu.VMEM((rows_per_ch, d), jnp.float32),
            pltpu.VMEM((rows_per_ch, d), jnp.float32),
            pltpu.SemaphoreType.DMA,
            pltpu.SemaphoreType.DMA,
            pltpu.SemaphoreType.DMA,
            pltpu.SemaphoreType.DMA,
        ],
    )
    def k(table_hbm, idx_hbm, out_hbm, idx_v, buf0, buf1, g0, g1, o0, o1):
        table_rows = table_hbm
        wid = lax.axis_index("s") * _NC + lax.axis_index("c")
        base = wid * per_w // pack
        pltpu.sync_copy(idx_hbm.at[wid], idx_v)
        bufs = (buf0, buf1)
        gsems = (g0, g1)
        osems = (o0, o1)

        def fire(c, b):
            for j in range(ch):
                pltpu.async_copy(
                    table_rows.at[idx_v.at[c * ch + j]],
                    bufs[b].at[pl.ds(j * _GSZ, _GSZ)],
                    gsems[b],
                )

        def drain_gathers(b):
            pltpu.make_async_copy(
                out_hbm.at[pl.ds(0, rows_per_ch)],
                bufs[b],
                gsems[b],
            ).wait()

        def out_copy(c, b):
            return pltpu.make_async_copy(
                bufs[b],
                out_hbm.at[pl.ds(base + c * rows_per_ch, rows_per_ch)],
                osems[b],
            )

        fire(0, 0)

        @pl.loop(0, n_ch // 2)
        def _(t):
            for b in range(2):
                c = t * 2 + b
                nb = 1 - b
                drain_gathers(b)
                out_copy(c, b).start()

                @pl.when(c >= 1)
                def _():
                    out_copy(c - 1, nb).wait()

                @pl.when(c + 1 < n_ch)
                def _():
                    fire(c + 1, nb)

        out_copy(n_ch - 1, (n_ch - 1) % 2).wait()

    return k(table2, idx3)


def kernel(x, table):
    if x.ndim != 4:
        raise ValueError(f"TokenEmbedding expects 4D input [B, H, W, C]. Got: {x.shape}")
    V, D = table.shape
    if x.shape[-1] == V:
        idx = jnp.argmax(x, axis=-1).astype(jnp.int32)
    else:
        idx = x.astype(jnp.int32)
    B, H, W = x.shape[0], x.shape[1], x.shape[2]
    N = B * H * W * (1 if x.shape[-1] == V else x.shape[-1])
    flat = idx.reshape(N)

    assert N % (_NW * _GSZ) == 0, (N,)
    assert 128 % D == 0, (D,)
    n_g = N // (_NW * _GSZ)
    ch = 1
    for cand in range(2, 9):
        if n_g % cand == 0 and (n_g // cand) % 2 == 0:
            ch = cand
    idx3 = flat.reshape(_NW, n_g, _GSZ)
    out = _sc_gather(table, idx3, n_g, ch, D)
    return out.reshape(B, H, W, D)
